# baseline (speedup 1.0000x reference)
import functools

import jax
import jax.numpy as jnp
import numpy as np
from jax.experimental import pallas as pl
from jax.experimental.pallas import tpu as pltpu

WINDOW = 2048
CHUNK = 512
K = WINDOW // CHUNK


def _window_start(max_len: int) -> int:
    if max_len == WINDOW:
        return 0
    return int(np.random.RandomState(0).randint(0, max_len - WINDOW + 1))


def _tc_body(start, nb, ddx, mdx, p, oddx_ref, omdx_ref, op_ref,
             bufd, bufm, bufp, semd, semm, semp):
    off = start % 8
    base = start - off
    b = pl.program_id(0)
    k = pl.program_id(1)
    g = b * K + k

    def start_dmas(bb, kk, slot):
        row = base + kk * CHUNK
        pltpu.make_async_copy(
            ddx.at[bb, pl.ds(row, CHUNK + 8)], bufd.at[slot], semd.at[slot]
        ).start()
        pltpu.make_async_copy(
            mdx.at[bb, pl.ds(row, CHUNK + 8)], bufm.at[slot], semm.at[slot]
        ).start()
        pltpu.make_async_copy(
            p.at[bb, pl.ds(row, CHUNK + 8)], bufp.at[slot], semp.at[slot]
        ).start()

    @pl.when(g == 0)
    def _():
        start_dmas(0, 0, 0)
        start_dmas(0, 1, 1)
        start_dmas(0, 2, 2)

    @pl.when(g + 3 < nb * K)
    def _():
        nk = (k + 3) % K
        nbb = b + (k + 3) // K
        start_dmas(nbb, nk, (g + 3) % 4)

    slot = g % 4
    row = base + k * CHUNK
    pltpu.make_async_copy(
        ddx.at[b, pl.ds(row, CHUNK + 8)], bufd.at[slot], semd.at[slot]
    ).wait()
    pltpu.make_async_copy(
        mdx.at[b, pl.ds(row, CHUNK + 8)], bufm.at[slot], semm.at[slot]
    ).wait()
    pltpu.make_async_copy(
        p.at[b, pl.ds(row, CHUNK + 8)], bufp.at[slot], semp.at[slot]
    ).wait()

    oddx_ref[0] = bufd[slot, pl.ds(off, CHUNK), :]
    omdx_ref[0] = bufm[slot, pl.ds(off, CHUNK), :]
    op_ref[0] = bufp[slot, pl.ds(off, CHUNK), :]


@jax.jit
def kernel(ddx, mdx, p):
    batch, max_len, dm = ddx.shape
    dp = p.shape[-1]
    start = _window_start(max_len)
    grid = (batch, K)
    out_shape = (
        jax.ShapeDtypeStruct((batch, WINDOW, dm), ddx.dtype),
        jax.ShapeDtypeStruct((batch, WINDOW, dm), mdx.dtype),
        jax.ShapeDtypeStruct((batch, WINDOW, dp), p.dtype),
    )
    return pl.pallas_call(
        functools.partial(_tc_body, start, batch),
        grid=grid,
        in_specs=[
            pl.BlockSpec(memory_space=pl.ANY),
            pl.BlockSpec(memory_space=pl.ANY),
            pl.BlockSpec(memory_space=pl.ANY),
        ],
        out_specs=(
            pl.BlockSpec((1, CHUNK, dm), lambda b, k: (b, k, 0)),
            pl.BlockSpec((1, CHUNK, dm), lambda b, k: (b, k, 0)),
            pl.BlockSpec((1, CHUNK, dp), lambda b, k: (b, k, 0)),
        ),
        out_shape=out_shape,
        scratch_shapes=[
            pltpu.VMEM((4, CHUNK + 8, dm), ddx.dtype),
            pltpu.VMEM((4, CHUNK + 8, dm), mdx.dtype),
            pltpu.VMEM((4, CHUNK + 8, dp), p.dtype),
            pltpu.SemaphoreType.DMA((4,)),
            pltpu.SemaphoreType.DMA((4,)),
            pltpu.SemaphoreType.DMA((4,)),
        ],
        compiler_params=pltpu.CompilerParams(
            dimension_semantics=("arbitrary", "arbitrary"),
        ),
    )(ddx, mdx, p)


# speedup vs baseline: 1.0041x; 1.0041x over previous
import functools

import jax
import jax.numpy as jnp
import numpy as np
from jax.experimental import pallas as pl
from jax.experimental.pallas import tpu as pltpu

WINDOW = 2048
CHUNK = 1024
K = WINDOW // CHUNK


def _window_start(max_len: int) -> int:
    if max_len == WINDOW:
        return 0
    return int(np.random.RandomState(0).randint(0, max_len - WINDOW + 1))


def _tc_body(start, nb, ddx, mdx, p, oddx_ref, omdx_ref, op_ref,
             bufd, bufm, bufp, semd, semm, semp):
    off = start % 8
    base = start - off
    b = pl.program_id(0)
    k = pl.program_id(1)
    g = b * K + k

    def start_dmas(bb, kk, slot):
        row = base + kk * CHUNK
        pltpu.make_async_copy(
            ddx.at[bb, pl.ds(row, CHUNK + 8)], bufd.at[slot], semd.at[slot]
        ).start()
        pltpu.make_async_copy(
            mdx.at[bb, pl.ds(row, CHUNK + 8)], bufm.at[slot], semm.at[slot]
        ).start()
        pltpu.make_async_copy(
            p.at[bb, pl.ds(row, CHUNK + 8)], bufp.at[slot], semp.at[slot]
        ).start()

    @pl.when(g == 0)
    def _():
        start_dmas(0, 0, 0)
        start_dmas(0, 1, 1)

    @pl.when(g + 2 < nb * K)
    def _():
        nk = (k + 2) % K
        nbb = b + (k + 2) // K
        start_dmas(nbb, nk, (g + 2) % 3)

    slot = g % 3
    row = base + k * CHUNK
    pltpu.make_async_copy(
        ddx.at[b, pl.ds(row, CHUNK + 8)], bufd.at[slot], semd.at[slot]
    ).wait()
    pltpu.make_async_copy(
        mdx.at[b, pl.ds(row, CHUNK + 8)], bufm.at[slot], semm.at[slot]
    ).wait()
    pltpu.make_async_copy(
        p.at[b, pl.ds(row, CHUNK + 8)], bufp.at[slot], semp.at[slot]
    ).wait()

    oddx_ref[0] = bufd[slot, pl.ds(off, CHUNK), :]
    omdx_ref[0] = bufm[slot, pl.ds(off, CHUNK), :]
    op_ref[0] = bufp[slot, pl.ds(off, CHUNK), :]


@jax.jit
def kernel(ddx, mdx, p):
    batch, max_len, dm = ddx.shape
    dp = p.shape[-1]
    start = _window_start(max_len)
    grid = (batch, K)
    out_shape = (
        jax.ShapeDtypeStruct((batch, WINDOW, dm), ddx.dtype),
        jax.ShapeDtypeStruct((batch, WINDOW, dm), mdx.dtype),
        jax.ShapeDtypeStruct((batch, WINDOW, dp), p.dtype),
    )
    return pl.pallas_call(
        functools.partial(_tc_body, start, batch),
        grid=grid,
        in_specs=[
            pl.BlockSpec(memory_space=pl.ANY),
            pl.BlockSpec(memory_space=pl.ANY),
            pl.BlockSpec(memory_space=pl.ANY),
        ],
        out_specs=(
            pl.BlockSpec((1, CHUNK, dm), lambda b, k: (b, k, 0)),
            pl.BlockSpec((1, CHUNK, dm), lambda b, k: (b, k, 0)),
            pl.BlockSpec((1, CHUNK, dp), lambda b, k: (b, k, 0)),
        ),
        out_shape=out_shape,
        scratch_shapes=[
            pltpu.VMEM((3, CHUNK + 8, dm), ddx.dtype),
            pltpu.VMEM((3, CHUNK + 8, dm), mdx.dtype),
            pltpu.VMEM((3, CHUNK + 8, dp), p.dtype),
            pltpu.SemaphoreType.DMA((3,)),
            pltpu.SemaphoreType.DMA((3,)),
            pltpu.SemaphoreType.DMA((3,)),
        ],
        compiler_params=pltpu.CompilerParams(
            dimension_semantics=("arbitrary", "arbitrary"),
        ),
    )(ddx, mdx, p)
